# baseline (speedup 1.0000x reference)
import functools

import jax
import jax.numpy as jnp
from jax import lax
from jax.experimental import pallas as pl
from jax.experimental.pallas import tpu as pltpu
from jax.experimental.pallas import tpu_sc as plsc

B, C, H, W = 16, 96, 224, 224
NPIX = H * W
TOTAL = B * NPIX
NC, NS, L = 2, 16, 16
NW = NC * NS
PER_W = TOTAL // NW
CHUNK = 64
NCHUNK = PER_W // CHUNK
HB = 16
NHB = H // HB
SCALE = 2.0 / (W - 1.0)
CP = 128


def _tc_body(x_ref, wl_ref, bl_ref, flat_ref, theta_ref, acc_ref):
    b = pl.program_id(0)
    i = pl.program_id(1)
    xb = x_ref[0].reshape(C, HB * W)
    xt = xb.T
    flat_ref[0, 0] = jnp.concatenate(
        [xt, jnp.zeros((HB * W, CP - C), jnp.float32)], axis=1)
    part = jnp.sum(xb, axis=1)

    @pl.when(i == 0)
    def _():
        acc_ref[...] = part

    @pl.when(i > 0)
    def _():
        acc_ref[...] = acc_ref[...] + part

    @pl.when(i == NHB - 1)
    def _():
        pooled = acc_ref[...] * jnp.float32(1.0 / NPIX)
        pooled_b = pooled.astype(jnp.bfloat16).astype(jnp.float32)
        wl_b = wl_ref[...].astype(jnp.bfloat16).astype(jnp.float32)
        th = jax.lax.dot_general(
            pooled_b.reshape(1, C), wl_b, (((1,), (0,)), ((), ())),
            preferred_element_type=jnp.float32,
            precision=jax.lax.Precision.HIGHEST)
        theta_ref[pl.ds(b, 1), :] = th + bl_ref[...]


def _stage_a(X, W_loc, b_loc):
    flat, theta = pl.pallas_call(
        _tc_body,
        grid=(B, NHB),
        in_specs=[
            pl.BlockSpec((1, C, HB, W), lambda b, i: (b, 0, i, 0)),
            pl.BlockSpec((C, 6), lambda b, i: (0, 0)),
            pl.BlockSpec((1, 6), lambda b, i: (0, 0)),
        ],
        out_specs=[
            pl.BlockSpec((1, 1, HB * W, CP), lambda b, i: (b, i, 0, 0)),
            pl.BlockSpec((B, 6), lambda b, i: (0, 0)),
        ],
        out_shape=[
            jax.ShapeDtypeStruct((B, NHB, HB * W, CP), jnp.float32),
            jax.ShapeDtypeStruct((B, 6), jnp.float32),
        ],
        scratch_shapes=[pltpu.VMEM((C,), jnp.float32)],
    )(X, W_loc, b_loc.reshape(1, 6))
    return flat.reshape(TOTAL, CP), theta.reshape(B * 6)


def _splat(ref, idx_scalar):
    return plsc.load_gather(ref, [jnp.full((L,), idx_scalar, jnp.int32)])


def _bf16r(v):
    u = plsc.bitcast(v, jnp.uint32)
    r = (u + jnp.uint32(0x7FFF) + ((u >> jnp.uint32(16)) & jnp.uint32(1)))
    r = r & jnp.uint32(0xFFFF0000)
    return plsc.bitcast(r, jnp.float32)


def _sc_body(flat_hbm, theta_hbm, out_hbm,
             theta_v, iac_v, ibd_v, wa_v, wb_v, wc_v, wd_v,
             gac_v, gbd_v, outt_v, sem0, sem1, sem2):
    wid = lax.axis_index("s") * NC + lax.axis_index("c")
    b = wid // 2
    half = wid % 2
    bbase = b * NPIX

    pltpu.sync_copy(theta_hbm, theta_v)
    t00 = _bf16r(_splat(theta_v, b * 6 + 0))
    t01 = _bf16r(_splat(theta_v, b * 6 + 1))
    t02 = _bf16r(_splat(theta_v, b * 6 + 2))
    t10 = _bf16r(_splat(theta_v, b * 6 + 3))
    t11 = _bf16r(_splat(theta_v, b * 6 + 4))
    t12 = _bf16r(_splat(theta_v, b * 6 + 5))
    lane = lax.iota(jnp.int32, L)
    ch_idx = [lane + k * L for k in range(6)]
    sems = (sem0, sem1, sem2)

    def compute_idx(g, buf):
        col0 = half * PER_W + g * CHUNK
        fb = jnp.full((L,), buf, jnp.int32)
        for v in range(CHUNK // L):
            pix = jnp.full((L,), col0 + v * L, jnp.int32) + lane
            irow = pix // W
            jcol = pix - irow * W
            xt = _bf16r(jcol.astype(jnp.float32) * SCALE - 1.0)
            yt = _bf16r(irow.astype(jnp.float32) * SCALE - 1.0)
            x = (t00 * xt + t01 * yt + t02 + 1.0) * jnp.float32(W / 2.0)
            y = (t10 * xt + t11 * yt + t12 + 1.0) * jnp.float32(H / 2.0)
            tx = x.astype(jnp.int32)
            x0u = jnp.where(tx.astype(jnp.float32) > x, tx - 1, tx)
            ty = y.astype(jnp.int32)
            y0u = jnp.where(ty.astype(jnp.float32) > y, ty - 1, ty)
            x0 = jnp.clip(x0u, 0, W - 1)
            x1 = jnp.clip(x0u + 1, 0, W - 1)
            y0 = jnp.clip(y0u, 0, H - 1)
            y1 = jnp.clip(y0u + 1, 0, H - 1)
            x0f = x0.astype(jnp.float32)
            x1f = x1.astype(jnp.float32)
            y0f = y0.astype(jnp.float32)
            y1f = y1.astype(jnp.float32)
            sl = pl.ds(v * L, L)
            wa_v[buf, sl] = (x1f - x) * (y1f - y)
            wb_v[buf, sl] = (x1f - x) * (y - y0f)
            wc_v[buf, sl] = (x - x0f) * (y1f - y)
            wd_v[buf, sl] = (x - x0f) * (y - y0f)
            row0 = bbase + y0 * W
            row1 = bbase + y1 * W
            pos = (lane + v * L) * 2
            plsc.store_scatter(iac_v, [fb, pos], row0 + x0)
            plsc.store_scatter(iac_v, [fb, pos + 1], row0 + x1)
            plsc.store_scatter(ibd_v, [fb, pos], row1 + x0)
            plsc.store_scatter(ibd_v, [fb, pos + 1], row1 + x1)

    def fire(buf):
        pltpu.make_async_copy(
            flat_hbm.at[iac_v.at[buf]], gac_v.at[buf], sems[buf]).start()
        pltpu.make_async_copy(
            flat_hbm.at[ibd_v.at[buf]], gbd_v.at[buf], sems[buf]).start()

    def drain(buf):
        pltpu.make_async_copy(
            flat_hbm.at[iac_v.at[buf]], gac_v.at[buf], sems[buf]).wait()
        pltpu.make_async_copy(
            flat_hbm.at[ibd_v.at[buf]], gbd_v.at[buf], sems[buf]).wait()

    def combine(g, buf):
        colbase = (g % 2) * CHUNK

        @plsc.parallel_loop(0, CHUNK, unroll=16)
        def _(r):
            fb = jnp.full((L,), buf, jnp.int32)
            fr = jnp.full((L,), r, jnp.int32)
            wa = plsc.load_gather(wa_v, [fb, fr])
            wb = plsc.load_gather(wb_v, [fb, fr])
            wc = plsc.load_gather(wc_v, [fb, fr])
            wd = plsc.load_gather(wd_v, [fb, fr])
            r2 = r * 2
            co = fr + jnp.full((L,), colbase, jnp.int32)
            for k in range(6):
                ck = pl.ds(k * L, L)
                acc = (wa * gac_v[buf, r2, ck] + wc * gac_v[buf, r2 + 1, ck]
                       + wb * gbd_v[buf, r2, ck] + wd * gbd_v[buf, r2 + 1, ck])
                plsc.store_scatter(outt_v, [ch_idx[k], co], acc)

    def flush(g):
        col0 = pl.multiple_of(half * PER_W + (g - 1) * CHUNK, 2 * CHUNK)
        pltpu.sync_copy(outt_v, out_hbm.at[pl.ds(b * C, C), pl.ds(col0, 2 * CHUNK)])

    compute_idx(0, 0)
    fire(0)
    compute_idx(1, 1)
    fire(1)

    def outer_body(o, carry):
        for p in range(3):
            g = o * 3 + p

            @pl.when(g + 2 < NCHUNK)
            def _():
                compute_idx(g + 2, (p + 2) % 3)
                fire((p + 2) % 3)

            drain(p)
            combine(g, p)

            @pl.when(g % 2 == 1)
            def _():
                flush(g)
        return carry

    lax.fori_loop(0, NCHUNK // 3, outer_body, 0, unroll=False)
    for g, p in ((NCHUNK - 2, (NCHUNK - 2) % 3), (NCHUNK - 1, (NCHUNK - 1) % 3)):
        drain(p)
        combine(g, p)
    flush(NCHUNK - 1)


@functools.cache
def _get_sc_call():
    return pl.kernel(
        _sc_body,
        out_type=jax.ShapeDtypeStruct((B * C, NPIX), jnp.float32),
        mesh=plsc.VectorSubcoreMesh(core_axis_name="c", subcore_axis_name="s",
                                    num_cores=NC, num_subcores=NS),
        compiler_params=pltpu.CompilerParams(needs_layout_passes=False),
        scratch_types=[
            pltpu.VMEM((B * 6,), jnp.float32),
            pltpu.VMEM((3, 2 * CHUNK), jnp.int32),
            pltpu.VMEM((3, 2 * CHUNK), jnp.int32),
            pltpu.VMEM((3, CHUNK), jnp.float32),
            pltpu.VMEM((3, CHUNK), jnp.float32),
            pltpu.VMEM((3, CHUNK), jnp.float32),
            pltpu.VMEM((3, CHUNK), jnp.float32),
            pltpu.VMEM((3, 2 * CHUNK, CP), jnp.float32),
            pltpu.VMEM((3, 2 * CHUNK, CP), jnp.float32),
            pltpu.VMEM((C, 2 * CHUNK), jnp.float32),
            pltpu.SemaphoreType.DMA,
            pltpu.SemaphoreType.DMA,
            pltpu.SemaphoreType.DMA,
        ],
    )


def kernel(X, W_loc, b_loc):
    flat, theta = _stage_a(X, W_loc, b_loc)
    out = _get_sc_call()(flat, theta)
    return out.reshape(B, C, H, W)


# speedup vs baseline: 1.0800x; 1.0800x over previous
import functools

import jax
import jax.numpy as jnp
from jax import lax
from jax.experimental import pallas as pl
from jax.experimental.pallas import tpu as pltpu
from jax.experimental.pallas import tpu_sc as plsc

B, C, H, W = 16, 96, 224, 224
NPIX = H * W
TOTAL = B * NPIX
NC, NS, L = 2, 16, 16
NW = NC * NS
PER_W = TOTAL // NW
CHUNK = 64
NCHUNK = PER_W // CHUNK
HB = 16
NHB = H // HB
SCALE = 2.0 / (W - 1.0)
CP = 128


def _tc_body(x_ref, wl_ref, bl_ref, flat_ref, theta_ref, acc_ref):
    b = pl.program_id(0)
    i = pl.program_id(1)
    xb = x_ref[0].reshape(C, HB * W)
    xt = xb.T
    flat_ref[0, 0] = jnp.concatenate(
        [xt, jnp.zeros((HB * W, CP - C), jnp.float32)], axis=1)
    part = jnp.sum(xb, axis=1)

    @pl.when(i == 0)
    def _():
        acc_ref[...] = part

    @pl.when(i > 0)
    def _():
        acc_ref[...] = acc_ref[...] + part

    @pl.when(i == NHB - 1)
    def _():
        pooled = acc_ref[...] * jnp.float32(1.0 / NPIX)
        pooled_b = pooled.astype(jnp.bfloat16).astype(jnp.float32)
        wl_b = wl_ref[...].astype(jnp.bfloat16).astype(jnp.float32)
        th = jax.lax.dot_general(
            pooled_b.reshape(1, C), wl_b, (((1,), (0,)), ((), ())),
            preferred_element_type=jnp.float32,
            precision=jax.lax.Precision.HIGHEST)
        theta_ref[pl.ds(b, 1), :] = th + bl_ref[...]


def _stage_a(X, W_loc, b_loc):
    flat, theta = pl.pallas_call(
        _tc_body,
        grid=(B, NHB),
        in_specs=[
            pl.BlockSpec((1, C, HB, W), lambda b, i: (b, 0, i, 0)),
            pl.BlockSpec((C, 6), lambda b, i: (0, 0)),
            pl.BlockSpec((1, 6), lambda b, i: (0, 0)),
        ],
        out_specs=[
            pl.BlockSpec((1, 1, HB * W, CP), lambda b, i: (b, i, 0, 0)),
            pl.BlockSpec((B, 6), lambda b, i: (0, 0)),
        ],
        out_shape=[
            jax.ShapeDtypeStruct((B, NHB, HB * W, CP), jnp.float32),
            jax.ShapeDtypeStruct((B, 6), jnp.float32),
        ],
        scratch_shapes=[pltpu.VMEM((C,), jnp.float32)],
    )(X, W_loc, b_loc.reshape(1, 6))
    return flat.reshape(TOTAL, CP), theta.reshape(B * 6)


def _splat(ref, idx_scalar):
    return plsc.load_gather(ref, [jnp.full((L,), idx_scalar, jnp.int32)])


def _bf16r(v):
    u = plsc.bitcast(v, jnp.uint32)
    r = (u + jnp.uint32(0x7FFF) + ((u >> jnp.uint32(16)) & jnp.uint32(1)))
    r = r & jnp.uint32(0xFFFF0000)
    return plsc.bitcast(r, jnp.float32)


def _sc_body(flat_hbm, theta_hbm, out_hbm,
             theta_v, iac_v, ibd_v, wa_v, wb_v, wc_v, wd_v,
             gac_v, gbd_v, outt_v, sem0, sem1, sem2):
    wid = lax.axis_index("s") * NC + lax.axis_index("c")
    b = wid // 2
    half = wid % 2
    bbase = b * NPIX

    pltpu.sync_copy(theta_hbm, theta_v)
    t00 = _bf16r(_splat(theta_v, b * 6 + 0))
    t01 = _bf16r(_splat(theta_v, b * 6 + 1))
    t02 = _bf16r(_splat(theta_v, b * 6 + 2))
    t10 = _bf16r(_splat(theta_v, b * 6 + 3))
    t11 = _bf16r(_splat(theta_v, b * 6 + 4))
    t12 = _bf16r(_splat(theta_v, b * 6 + 5))
    lane = lax.iota(jnp.int32, L)
    ch_idx = [lane + k * L for k in range(6)]
    sems = (sem0, sem1, sem2)

    def compute_idx(g, buf):
        col0 = half * PER_W + g * CHUNK
        fb = jnp.full((L,), buf, jnp.int32)
        for v in range(CHUNK // L):
            pix = jnp.full((L,), col0 + v * L, jnp.int32) + lane
            irow = pix // W
            jcol = pix - irow * W
            xt = _bf16r(jcol.astype(jnp.float32) * SCALE - 1.0)
            yt = _bf16r(irow.astype(jnp.float32) * SCALE - 1.0)
            x = (t00 * xt + t01 * yt + t02 + 1.0) * jnp.float32(W / 2.0)
            y = (t10 * xt + t11 * yt + t12 + 1.0) * jnp.float32(H / 2.0)
            tx = x.astype(jnp.int32)
            x0u = jnp.where(tx.astype(jnp.float32) > x, tx - 1, tx)
            ty = y.astype(jnp.int32)
            y0u = jnp.where(ty.astype(jnp.float32) > y, ty - 1, ty)
            x0 = jnp.clip(x0u, 0, W - 1)
            x1 = jnp.clip(x0u + 1, 0, W - 1)
            y0 = jnp.clip(y0u, 0, H - 1)
            y1 = jnp.clip(y0u + 1, 0, H - 1)
            x0f = x0.astype(jnp.float32)
            x1f = x1.astype(jnp.float32)
            y0f = y0.astype(jnp.float32)
            y1f = y1.astype(jnp.float32)
            sl = pl.ds(v * L, L)
            wa_v[buf, sl] = (x1f - x) * (y1f - y)
            wb_v[buf, sl] = (x1f - x) * (y - y0f)
            wc_v[buf, sl] = (x - x0f) * (y1f - y)
            wd_v[buf, sl] = (x - x0f) * (y - y0f)
            row0 = bbase + y0 * W
            row1 = bbase + y1 * W
            pos = (lane + v * L) * 2
            plsc.store_scatter(iac_v, [fb, pos], row0 + x0)
            plsc.store_scatter(iac_v, [fb, pos + 1], row0 + x1)
            plsc.store_scatter(ibd_v, [fb, pos], row1 + x0)
            plsc.store_scatter(ibd_v, [fb, pos + 1], row1 + x1)

    def fire(buf):
        pltpu.make_async_copy(
            flat_hbm.at[iac_v.at[buf]], gac_v.at[buf], sems[buf]).start()
        pltpu.make_async_copy(
            flat_hbm.at[ibd_v.at[buf]], gbd_v.at[buf], sems[buf]).start()

    def drain(buf):
        pltpu.make_async_copy(
            flat_hbm.at[iac_v.at[buf]], gac_v.at[buf], sems[buf]).wait()
        pltpu.make_async_copy(
            flat_hbm.at[ibd_v.at[buf]], gbd_v.at[buf], sems[buf]).wait()

    def combine(g, buf):
        colbase = (g % 2) * CHUNK

        @plsc.parallel_loop(0, CHUNK, unroll=4)
        def _(r):
            fb = jnp.full((L,), buf, jnp.int32)
            fr = jnp.full((L,), r, jnp.int32)
            wa = plsc.load_gather(wa_v, [fb, fr])
            wb = plsc.load_gather(wb_v, [fb, fr])
            wc = plsc.load_gather(wc_v, [fb, fr])
            wd = plsc.load_gather(wd_v, [fb, fr])
            r2 = r * 2
            co = fr + jnp.full((L,), colbase, jnp.int32)
            for k in range(6):
                ck = pl.ds(k * L, L)
                acc = (wa * gac_v[buf, r2, ck] + wc * gac_v[buf, r2 + 1, ck]
                       + wb * gbd_v[buf, r2, ck] + wd * gbd_v[buf, r2 + 1, ck])
                plsc.store_scatter(outt_v, [ch_idx[k], co], acc)

    def flush(g):
        col0 = pl.multiple_of(half * PER_W + (g - 1) * CHUNK, 2 * CHUNK)
        pltpu.sync_copy(outt_v.at[:, pl.ds(0, 2 * CHUNK)],
                        out_hbm.at[pl.ds(b * C, C), pl.ds(col0, 2 * CHUNK)])

    compute_idx(0, 0)
    fire(0)
    compute_idx(1, 1)
    fire(1)

    def outer_body(o, carry):
        for p in range(3):
            g = o * 3 + p

            @pl.when(g + 2 < NCHUNK)
            def _():
                compute_idx(g + 2, (p + 2) % 3)
                fire((p + 2) % 3)

            drain(p)
            combine(g, p)

            @pl.when(g % 2 == 1)
            def _():
                flush(g)
        return carry

    lax.fori_loop(0, NCHUNK // 3, outer_body, 0, unroll=False)
    for g, p in ((NCHUNK - 2, (NCHUNK - 2) % 3), (NCHUNK - 1, (NCHUNK - 1) % 3)):
        drain(p)
        combine(g, p)
    flush(NCHUNK - 1)


@functools.cache
def _get_sc_call():
    return pl.kernel(
        _sc_body,
        out_type=jax.ShapeDtypeStruct((B * C, NPIX), jnp.float32),
        mesh=plsc.VectorSubcoreMesh(core_axis_name="c", subcore_axis_name="s",
                                    num_cores=NC, num_subcores=NS),
        compiler_params=pltpu.CompilerParams(needs_layout_passes=False),
        scratch_types=[
            pltpu.VMEM((B * 6,), jnp.float32),
            pltpu.VMEM((3, 2 * CHUNK), jnp.int32),
            pltpu.VMEM((3, 2 * CHUNK), jnp.int32),
            pltpu.VMEM((3, CHUNK), jnp.float32),
            pltpu.VMEM((3, CHUNK), jnp.float32),
            pltpu.VMEM((3, CHUNK), jnp.float32),
            pltpu.VMEM((3, CHUNK), jnp.float32),
            pltpu.VMEM((3, 2 * CHUNK, CP), jnp.float32),
            pltpu.VMEM((3, 2 * CHUNK, CP), jnp.float32),
            pltpu.VMEM((C, 2 * CHUNK + 1), jnp.float32),
            pltpu.SemaphoreType.DMA,
            pltpu.SemaphoreType.DMA,
            pltpu.SemaphoreType.DMA,
        ],
    )


def kernel(X, W_loc, b_loc):
    flat, theta = _stage_a(X, W_loc, b_loc)
    out = _get_sc_call()(flat, theta)
    return out.reshape(B, C, H, W)


# speedup vs baseline: 1.0983x; 1.0170x over previous
import functools

import jax
import jax.numpy as jnp
from jax import lax
from jax.experimental import pallas as pl
from jax.experimental.pallas import tpu as pltpu
from jax.experimental.pallas import tpu_sc as plsc

B, C, H, W = 16, 96, 224, 224
NPIX = H * W
TOTAL = B * NPIX
NC, NS, L = 2, 16, 16
NW = NC * NS
PER_W = TOTAL // NW
CHUNK = 64
NCHUNK = PER_W // CHUNK
HB = 16
NHB = H // HB
SCALE = 2.0 / (W - 1.0)
CP = 128


def _tc_body(x_ref, wl_ref, bl_ref, flat_ref, theta_ref, acc_ref):
    b = pl.program_id(0)
    i = pl.program_id(1)
    xb = x_ref[0].reshape(C, HB * W)
    xt = xb.T
    flat_ref[0, 0] = jnp.concatenate(
        [xt, jnp.zeros((HB * W, CP - C), jnp.float32)], axis=1)
    part = jnp.sum(xb, axis=1)

    @pl.when(i == 0)
    def _():
        acc_ref[...] = part

    @pl.when(i > 0)
    def _():
        acc_ref[...] = acc_ref[...] + part

    @pl.when(i == NHB - 1)
    def _():
        pooled = acc_ref[...] * jnp.float32(1.0 / NPIX)
        pooled_b = pooled.astype(jnp.bfloat16).astype(jnp.float32)
        wl_b = wl_ref[...].astype(jnp.bfloat16).astype(jnp.float32)
        th = jax.lax.dot_general(
            pooled_b.reshape(1, C), wl_b, (((1,), (0,)), ((), ())),
            preferred_element_type=jnp.float32,
            precision=jax.lax.Precision.HIGHEST)
        theta_ref[pl.ds(b, 1), :] = th + bl_ref[...]


def _stage_a(X, W_loc, b_loc):
    flat, theta = pl.pallas_call(
        _tc_body,
        grid=(B, NHB),
        in_specs=[
            pl.BlockSpec((1, C, HB, W), lambda b, i: (b, 0, i, 0)),
            pl.BlockSpec((C, 6), lambda b, i: (0, 0)),
            pl.BlockSpec((1, 6), lambda b, i: (0, 0)),
        ],
        out_specs=[
            pl.BlockSpec((1, 1, HB * W, CP), lambda b, i: (b, i, 0, 0)),
            pl.BlockSpec((B, 6), lambda b, i: (0, 0)),
        ],
        out_shape=[
            jax.ShapeDtypeStruct((B, NHB, HB * W, CP), jnp.float32),
            jax.ShapeDtypeStruct((B, 6), jnp.float32),
        ],
        scratch_shapes=[pltpu.VMEM((C,), jnp.float32)],
    )(X, W_loc, b_loc.reshape(1, 6))
    return flat.reshape(TOTAL, CP), theta.reshape(B * 6)


def _splat(ref, idx_scalar):
    return plsc.load_gather(ref, [jnp.full((L,), idx_scalar, jnp.int32)])


def _bf16r(v):
    u = plsc.bitcast(v, jnp.uint32)
    r = (u + jnp.uint32(0x7FFF) + ((u >> jnp.uint32(16)) & jnp.uint32(1)))
    r = r & jnp.uint32(0xFFFF0000)
    return plsc.bitcast(r, jnp.float32)


def _sc_body(flat_hbm, theta_hbm, out_hbm,
             theta_v, iac_v, ibd_v, wa_v, wb_v, wc_v, wd_v,
             gac_v, gbd_v, outt_v, sem0, sem1):
    wid = lax.axis_index("s") * NC + lax.axis_index("c")
    b = wid // 2
    half = wid % 2
    bbase = b * NPIX

    pltpu.sync_copy(theta_hbm, theta_v)
    t00 = _bf16r(_splat(theta_v, b * 6 + 0))
    t01 = _bf16r(_splat(theta_v, b * 6 + 1))
    t02 = _bf16r(_splat(theta_v, b * 6 + 2))
    t10 = _bf16r(_splat(theta_v, b * 6 + 3))
    t11 = _bf16r(_splat(theta_v, b * 6 + 4))
    t12 = _bf16r(_splat(theta_v, b * 6 + 5))
    lane = lax.iota(jnp.int32, L)
    ch_idx = [lane + k * L for k in range(6)]
    sems = (sem0, sem1)

    def compute_idx(g, buf):
        col0 = half * PER_W + g * CHUNK
        fb = jnp.full((L,), buf, jnp.int32)
        for v in range(CHUNK // L):
            pix = jnp.full((L,), col0 + v * L, jnp.int32) + lane
            irow = pix // W
            jcol = pix - irow * W
            xt = _bf16r(jcol.astype(jnp.float32) * SCALE - 1.0)
            yt = _bf16r(irow.astype(jnp.float32) * SCALE - 1.0)
            x = (t00 * xt + t01 * yt + t02 + 1.0) * jnp.float32(W / 2.0)
            y = (t10 * xt + t11 * yt + t12 + 1.0) * jnp.float32(H / 2.0)
            tx = x.astype(jnp.int32)
            x0u = jnp.where(tx.astype(jnp.float32) > x, tx - 1, tx)
            ty = y.astype(jnp.int32)
            y0u = jnp.where(ty.astype(jnp.float32) > y, ty - 1, ty)
            x0 = jnp.clip(x0u, 0, W - 1)
            x1 = jnp.clip(x0u + 1, 0, W - 1)
            y0 = jnp.clip(y0u, 0, H - 1)
            y1 = jnp.clip(y0u + 1, 0, H - 1)
            x0f = x0.astype(jnp.float32)
            x1f = x1.astype(jnp.float32)
            y0f = y0.astype(jnp.float32)
            y1f = y1.astype(jnp.float32)
            sl = pl.ds(v * L, L)
            wa_v[buf, sl] = (x1f - x) * (y1f - y)
            wb_v[buf, sl] = (x1f - x) * (y - y0f)
            wc_v[buf, sl] = (x - x0f) * (y1f - y)
            wd_v[buf, sl] = (x - x0f) * (y - y0f)
            row0 = bbase + y0 * W
            row1 = bbase + y1 * W
            pos = (lane + v * L) * 2
            plsc.store_scatter(iac_v, [fb, pos], row0 + x0)
            plsc.store_scatter(iac_v, [fb, pos + 1], row0 + x1)
            plsc.store_scatter(ibd_v, [fb, pos], row1 + x0)
            plsc.store_scatter(ibd_v, [fb, pos + 1], row1 + x1)

    def fire(buf):
        pltpu.make_async_copy(
            flat_hbm.at[iac_v.at[buf]], gac_v.at[buf], sems[buf]).start()
        pltpu.make_async_copy(
            flat_hbm.at[ibd_v.at[buf]], gbd_v.at[buf], sems[buf]).start()

    def drain(buf):
        pltpu.make_async_copy(
            flat_hbm.at[iac_v.at[buf]], gac_v.at[buf], sems[buf]).wait()
        pltpu.make_async_copy(
            flat_hbm.at[ibd_v.at[buf]], gbd_v.at[buf], sems[buf]).wait()

    def combine(buf):
        @plsc.parallel_loop(0, CHUNK, unroll=4)
        def _(r):
            fb = jnp.full((L,), buf, jnp.int32)
            fr = jnp.full((L,), r, jnp.int32)
            wa = plsc.load_gather(wa_v, [fb, fr])
            wb = plsc.load_gather(wb_v, [fb, fr])
            wc = plsc.load_gather(wc_v, [fb, fr])
            wd = plsc.load_gather(wd_v, [fb, fr])
            r2 = r * 2
            for k in range(6):
                ck = pl.ds(k * L, L)
                acc = (wa * gac_v[buf, r2, ck] + wc * gac_v[buf, r2 + 1, ck]
                       + wb * gbd_v[buf, r2, ck] + wd * gbd_v[buf, r2 + 1, ck])
                plsc.store_scatter(outt_v, [ch_idx[k], fr + buf * CHUNK], acc)

    compute_idx(0, 0)
    fire(0)

    def outer_body(o, carry):
        for p in (0, 1):
            g = o * 2 + p

            @pl.when(g + 1 < NCHUNK)
            def _():
                compute_idx(g + 1, 1 - p)
                fire(1 - p)

            drain(p)
            combine(p)
        col0 = half * PER_W + o * (2 * CHUNK)
        pltpu.sync_copy(outt_v, out_hbm.at[pl.ds(b * C, C), pl.ds(col0, 2 * CHUNK)])
        return carry

    lax.fori_loop(0, NCHUNK // 2, outer_body, 0, unroll=False)


@functools.cache
def _get_sc_call():
    return pl.kernel(
        _sc_body,
        out_type=jax.ShapeDtypeStruct((B * C, NPIX), jnp.float32),
        mesh=plsc.VectorSubcoreMesh(core_axis_name="c", subcore_axis_name="s",
                                    num_cores=NC, num_subcores=NS),
        compiler_params=pltpu.CompilerParams(needs_layout_passes=False),
        scratch_types=[
            pltpu.VMEM((B * 6,), jnp.float32),
            pltpu.VMEM((2, 2 * CHUNK), jnp.int32),
            pltpu.VMEM((2, 2 * CHUNK), jnp.int32),
            pltpu.VMEM((2, CHUNK), jnp.float32),
            pltpu.VMEM((2, CHUNK), jnp.float32),
            pltpu.VMEM((2, CHUNK), jnp.float32),
            pltpu.VMEM((2, CHUNK), jnp.float32),
            pltpu.VMEM((2, 2 * CHUNK, CP), jnp.float32),
            pltpu.VMEM((2, 2 * CHUNK, CP), jnp.float32),
            pltpu.VMEM((C, 2 * CHUNK), jnp.float32),
            pltpu.SemaphoreType.DMA,
            pltpu.SemaphoreType.DMA,
        ],
    )


def kernel(X, W_loc, b_loc):
    flat, theta = _stage_a(X, W_loc, b_loc)
    out = _get_sc_call()(flat, theta)
    return out.reshape(B, C, H, W)
